# initial kernel scaffold (unmeasured)
import jax
import jax.numpy as jnp
from jax import lax
from jax.experimental import pallas as pl
from jax.experimental.pallas import tpu as pltpu


def kernel(ids, E):
    T = ids.shape[0]
    V, D = E.shape

    def body(ids_ref, e_ref, out_ref, comm_ref, send_sem, recv_sem):
        my_x = lax.axis_index("x")
        my_y = lax.axis_index("y")
        my_z = lax.axis_index("z")
        peer = (1 - my_x, my_y, my_z)

        barrier = pltpu.get_barrier_semaphore()
        pl.semaphore_signal(
            barrier, inc=1, device_id=peer, device_id_type=pl.DeviceIdType.MESH
        )
        pl.semaphore_wait(barrier, 1)

        base = my_x * V

        def gather_row(t, _):
            idx = ids_ref[t] - base
            valid = jnp.logical_and(idx >= 0, idx < V)
            safe = jnp.where(valid, idx, 0)
            row = pl.load(e_ref, (pl.ds(safe, 1), slice(None)))
            scale = jnp.where(valid, 1.0, 0.0).astype(row.dtype)
            pl.store(
                comm_ref,
                (0, pl.ds(t, 1), slice(None)),
                (row * scale).astype(comm_ref.dtype),
            )
            return 0

        lax.fori_loop(0, T, gather_row, 0)

        rdma = pltpu.make_async_remote_copy(
            src_ref=comm_ref.at[0],
            dst_ref=comm_ref.at[1],
            send_sem=send_sem,
            recv_sem=recv_sem,
            device_id=peer,
            device_id_type=pl.DeviceIdType.MESH,
        )
        rdma.start()
        rdma.wait()

        out_ref[...] = comm_ref[0].astype(jnp.float32) + comm_ref[1].astype(
            jnp.float32
        )

    return pl.pallas_call(
        body,
        out_shape=jax.ShapeDtypeStruct((T, D), jnp.float32),
        in_specs=[
            pl.BlockSpec(memory_space=pltpu.SMEM),
            pl.BlockSpec(memory_space=pltpu.VMEM),
        ],
        out_specs=pl.BlockSpec(memory_space=pltpu.VMEM),
        scratch_shapes=[
            pltpu.VMEM((2, T, D), jnp.bfloat16),
            pltpu.SemaphoreType.DMA,
            pltpu.SemaphoreType.DMA,
        ],
        compiler_params=pltpu.CompilerParams(collective_id=0),
    )(ids, E)


# baseline (device time: 50835 ns/iter reference)
import jax
import jax.numpy as jnp
from jax import lax
from jax.experimental import pallas as pl
from jax.experimental.pallas import tpu as pltpu


def kernel(ids, E):
    T = ids.shape[0]
    V, D = E.shape

    def body(ids_ref, e_ref, out_ref, comm_ref, send_sem, recv_sem):
        my_x = lax.axis_index("x")
        my_y = lax.axis_index("y")
        my_z = lax.axis_index("z")
        peer = (1 - my_x, my_y, my_z)

        barrier = pltpu.get_barrier_semaphore()
        pl.semaphore_signal(
            barrier, inc=1, device_id=peer, device_id_type=pl.DeviceIdType.MESH
        )
        pl.semaphore_wait(barrier, 1)

        base = my_x * V

        GRP = 8

        def gather_group(g, _):
            rows = []
            for j in range(GRP):
                idx = ids_ref[g * GRP + j] - base
                valid = jnp.logical_and(idx >= 0, idx < V)
                safe = jnp.where(valid, idx, 0)
                row = e_ref[pl.ds(safe, 1), :]
                scale = jnp.where(valid, 1.0, 0.0).astype(row.dtype)
                rows.append((row * scale).astype(comm_ref.dtype))
            block = jnp.concatenate(rows, axis=0)
            comm_ref[0, pl.ds(pl.multiple_of(g * GRP, GRP), GRP), :] = block
            return 0

        lax.fori_loop(0, T // GRP, gather_group, 0)

        rdma = pltpu.make_async_remote_copy(
            src_ref=comm_ref.at[0],
            dst_ref=comm_ref.at[1],
            send_sem=send_sem,
            recv_sem=recv_sem,
            device_id=peer,
            device_id_type=pl.DeviceIdType.MESH,
        )
        rdma.start()
        rdma.wait()

        out_ref[...] = comm_ref[0].astype(jnp.float32) + comm_ref[1].astype(
            jnp.float32
        )

    return pl.pallas_call(
        body,
        out_shape=jax.ShapeDtypeStruct((T, D), jnp.float32),
        in_specs=[
            pl.BlockSpec(memory_space=pltpu.SMEM),
            pl.BlockSpec(memory_space=pltpu.VMEM),
        ],
        out_specs=pl.BlockSpec(memory_space=pltpu.VMEM),
        scratch_shapes=[
            pltpu.VMEM((2, T, D), jnp.bfloat16),
            pltpu.SemaphoreType.DMA,
            pltpu.SemaphoreType.DMA,
        ],
        compiler_params=pltpu.CompilerParams(
            collective_id=0, vmem_limit_bytes=100 * 1024 * 1024
        ),
    )(ids, E)


# device time: 26516 ns/iter; 1.9171x vs baseline; 1.9171x over previous
import os

import jax
import jax.numpy as jnp
from jax import lax
from jax.experimental import pallas as pl
from jax.experimental.pallas import tpu as pltpu

NC = 8
SLABS = 4
DIAG = os.environ.get("KERNEL_DIAG", "")


def kernel(ids, E):
    T = ids.shape[0]
    V, D = E.shape
    QC = D // 4
    CR = T // NC
    GRP = 8
    H = NC // 2

    def body(
        ids_ref,
        e_hbm,
        out_ref,
        e_vmem,
        part,
        recvx,
        red,
        recv_yown,
        recv_zown,
        recv_dy,
        recv_dz,
        e_sems,
        xs_send,
        xs_recv,
        yo_send,
        yo_recv,
        zo_send,
        zo_recv,
        yf_send,
        yf_recv,
        zf_send,
        zf_recv,
    ):
        my_x = lax.axis_index("x")
        my_y = lax.axis_index("y")
        my_z = lax.axis_index("z")
        zm = lax.rem(my_z, 2)
        xpeer = (1 - my_x, my_y, my_z)
        ypeer = (my_x, 1 - my_y, my_z)
        zpeer = (my_x, my_y, my_z + 1 - 2 * zm)

        q_mine = 2 * my_y + zm
        q_y = 2 * (1 - my_y) + zm
        q_z = 2 * my_y + (1 - zm)
        q_d = 2 * (1 - my_y) + (1 - zm)

        def colq(q):
            return pl.ds(pl.multiple_of(q * QC, QC), QC)

        SR = V // SLABS
        edmas = []
        for s in range(SLABS):
            cp = pltpu.make_async_copy(
                e_hbm.at[pl.ds(s * SR, SR), colq(q_mine)],
                e_vmem.at[pl.ds(s * SR, SR), :],
                e_sems.at[s],
            )
            cp.start()
            edmas.append(cp)

        if not DIAG:
            barrier = pltpu.get_barrier_semaphore()
            for nbr in (xpeer, ypeer, zpeer):
                pl.semaphore_signal(
                    barrier, inc=1, device_id=nbr, device_id_type=pl.DeviceIdType.MESH
                )
            pl.semaphore_wait(barrier, 3)

        for cp in edmas:
            cp.wait()

        if DIAG == "prefetch":
            out_ref[:, 0:QC] = e_vmem[0:T, :].astype(out_ref.dtype)

        base = my_x * V

        def rdma(src, dst, ssem, rsem, dev):
            return pltpu.make_async_remote_copy(
                src_ref=src,
                dst_ref=dst,
                send_sem=ssem,
                recv_sem=rsem,
                device_id=dev,
                device_id_type=pl.DeviceIdType.MESH,
            )

        x_rdmas = []
        for c in [] if DIAG == "prefetch" else range(NC):
            def gather_group(g, _, c=c):
                rows = []
                for j in range(GRP):
                    idx = ids_ref[c * CR + g * GRP + j] - base
                    valid = jnp.logical_and(idx >= 0, idx < V)
                    safe = jnp.where(valid, idx, 0)
                    row = e_vmem[pl.ds(safe, 1), :]
                    scale = jnp.where(valid, 1.0, 0.0).astype(row.dtype)
                    rows.append((row * scale).astype(part.dtype))
                block = jnp.concatenate(rows, axis=0)
                part[pl.ds(pl.multiple_of(c * CR + g * GRP, GRP), GRP), :] = block
                return 0

            lax.fori_loop(0, CR // GRP, gather_group, 0)
            if not DIAG:
                rsl = pl.ds(c * CR, CR)
                rx = rdma(
                    part.at[rsl, :],
                    recvx.at[rsl, :],
                    xs_send.at[c],
                    xs_recv.at[c],
                    xpeer,
                )
                rx.start()
                x_rdmas.append(rx)

        if DIAG == "nocomm":
            out_ref[:, 0:QC] = part[:, :]

        y_own, z_own = [], []
        for c in [] if DIAG else range(NC):
            x_rdmas[c].wait_recv()
            rsl = pl.ds(c * CR, CR)
            red[rsl, :] = part[rsl, :] + recvx[rsl, :]
            ry = rdma(
                red.at[rsl, :],
                recv_yown.at[rsl, :],
                yo_send.at[c],
                yo_recv.at[c],
                ypeer,
            )
            ry.start()
            y_own.append(ry)
            rz = rdma(
                red.at[rsl, :],
                recv_zown.at[rsl, :],
                zo_send.at[c],
                zo_recv.at[c],
                zpeer,
            )
            rz.start()
            z_own.append(rz)
            out_ref[rsl, colq(q_mine)] = red[rsl, :]

        y_fwd, z_fwd = [], []
        for c in [] if DIAG else range(H):
            z_own[c].wait_recv()
            rsl = pl.ds(c * CR, CR)
            out_ref[rsl, colq(q_z)] = recv_zown[rsl, :]
            rf = rdma(
                recv_zown.at[rsl, :],
                recv_dy.at[rsl, :],
                yf_send.at[c],
                yf_recv.at[c],
                ypeer,
            )
            rf.start()
            y_fwd.append(rf)
        for c in [] if DIAG else range(H, NC):
            y_own[c].wait_recv()
            rsl = pl.ds(c * CR, CR)
            out_ref[rsl, colq(q_y)] = recv_yown[rsl, :]
            rf = rdma(
                recv_yown.at[rsl, :],
                recv_dz.at[pl.ds((c - H) * CR, CR), :],
                zf_send.at[c - H],
                zf_recv.at[c - H],
                zpeer,
            )
            rf.start()
            z_fwd.append(rf)

        for c in [] if DIAG else range(H, NC):
            z_own[c].wait_recv()
            rsl = pl.ds(c * CR, CR)
            out_ref[rsl, colq(q_z)] = recv_zown[rsl, :]
        for c in [] if DIAG else range(H):
            y_own[c].wait_recv()
            rsl = pl.ds(c * CR, CR)
            out_ref[rsl, colq(q_y)] = recv_yown[rsl, :]

        for c in [] if DIAG else range(H):
            y_fwd[c].wait_recv()
            rsl = pl.ds(c * CR, CR)
            out_ref[rsl, colq(q_d)] = recv_dy[rsl, :]
        for c in [] if DIAG else range(H):
            z_fwd[c].wait_recv()
            out_ref[pl.ds((c + H) * CR, CR), colq(q_d)] = recv_dz[
                pl.ds(c * CR, CR), :
            ]

        for r in x_rdmas + y_own + z_own + y_fwd + z_fwd:
            r.wait_send()

    return pl.pallas_call(
        body,
        out_shape=jax.ShapeDtypeStruct((T, D), jnp.bfloat16),
        in_specs=[
            pl.BlockSpec(memory_space=pltpu.SMEM),
            pl.BlockSpec(memory_space=pl.ANY),
        ],
        out_specs=pl.BlockSpec(memory_space=pltpu.VMEM),
        scratch_shapes=[
            pltpu.VMEM((V, QC), jnp.float32),
            pltpu.VMEM((T, QC), jnp.bfloat16),
            pltpu.VMEM((T, QC), jnp.bfloat16),
            pltpu.VMEM((T, QC), jnp.bfloat16),
            pltpu.VMEM((T, QC), jnp.bfloat16),
            pltpu.VMEM((T, QC), jnp.bfloat16),
            pltpu.VMEM((T // 2, QC), jnp.bfloat16),
            pltpu.VMEM((T // 2, QC), jnp.bfloat16),
            pltpu.SemaphoreType.DMA((SLABS,)),
            pltpu.SemaphoreType.DMA((NC,)),
            pltpu.SemaphoreType.DMA((NC,)),
            pltpu.SemaphoreType.DMA((NC,)),
            pltpu.SemaphoreType.DMA((NC,)),
            pltpu.SemaphoreType.DMA((NC,)),
            pltpu.SemaphoreType.DMA((NC,)),
            pltpu.SemaphoreType.DMA((NC // 2,)),
            pltpu.SemaphoreType.DMA((NC // 2,)),
            pltpu.SemaphoreType.DMA((NC // 2,)),
            pltpu.SemaphoreType.DMA((NC // 2,)),
        ],
        compiler_params=pltpu.CompilerParams(
            collective_id=None if DIAG else 0,
            vmem_limit_bytes=100 * 1024 * 1024,
        ),
    )(ids, E)
